# baseline (speedup 1.0000x reference)
import functools

import jax
import jax.numpy as jnp
from jax import lax
from jax.experimental import pallas as pl
from jax.experimental.pallas import tpu as pltpu
from jax.experimental.pallas import tpu_sc as plsc

N = 10000
E = 320000
D = 128

NC = 2
NS = 16
CHUNK = 128
CPT = 80
ROWS2D = NC * NS * CPT
E_PAD = ROWS2D * CHUNK
N_ACC = 10112
ZROWS = N_ACC // NS
NBUF = 2
NPH = 2
HC = CPT // NPH


def _sc_aggregate(h, row2d, col2d, zslab):
    mesh = plsc.VectorSubcoreMesh(core_axis_name="c", subcore_axis_name="s")

    @functools.partial(
        pl.kernel,
        out_type=jax.ShapeDtypeStruct((NC, N_ACC, D), jnp.float32),
        mesh=mesh,
        scratch_types=(
            [pltpu.VMEM_SHARED((N_ACC, D), jnp.float32)]
            + [pltpu.VMEM((HC, CHUNK), jnp.int32)] * 2
            + [pltpu.VMEM((CHUNK, D), jnp.float32)] * NBUF
            + [pltpu.SemaphoreType.DMA] * (NBUF + 1)
        ),
    )
    def agg(h_hbm, row_hbm, col_hbm, z_hbm, out_hbm, acc, row_v, col_v, *rest):
        bufs = rest[:NBUF]
        gs = rest[NBUF:2 * NBUF]
        zs = rest[2 * NBUF]
        c = lax.axis_index("c")
        s = lax.axis_index("s")
        base = (c * NS + s) * CPT
        pltpu.async_copy(z_hbm, acc.at[pl.ds(s * ZROWS, ZROWS)], zs)
        pltpu.async_copy(row_hbm.at[pl.ds(base, HC)], row_v, gs[0])
        pltpu.async_copy(col_hbm.at[pl.ds(base, HC)], col_v, gs[1])
        pltpu.make_async_copy(z_hbm, acc.at[pl.ds(s * ZROWS, ZROWS)], zs).wait()
        pltpu.make_async_copy(row_hbm.at[pl.ds(base, HC)], row_v, gs[0]).wait()
        pltpu.make_async_copy(col_hbm.at[pl.ds(base, HC)], col_v, gs[1]).wait()
        plsc.subcore_barrier()

        for p in range(NPH):
            if p > 0:
                pltpu.sync_copy(row_hbm.at[pl.ds(base + p * HC, HC)], row_v)
                pltpu.sync_copy(col_hbm.at[pl.ds(base + p * HC, HC)], col_v)

            for b in range(NBUF):
                pltpu.async_copy(h_hbm.at[row_v.at[b]], bufs[b], gs[b])

            @pl.loop(0, HC // NBUF)
            def _(i):
                jbase = i * NBUF
                for b in range(NBUF):
                    j = jbase + b
                    pltpu.make_async_copy(
                        h_hbm.at[row_v.at[j]], bufs[b], gs[b]).wait()
                    pltpu.sync_copy(bufs[b], acc.at[col_v.at[j]], add=True)
                    jn = jbase + NBUF + b

                    @pl.when(jn < HC)
                    def _(b=b, jn=jn):
                        pltpu.async_copy(h_hbm.at[row_v.at[jn]], bufs[b], gs[b])

        plsc.subcore_barrier()
        pltpu.sync_copy(acc.at[pl.ds(s * ZROWS, ZROWS)],
                        out_hbm.at[c, pl.ds(s * ZROWS, ZROWS)])

    return agg(h, row2d, col2d, zslab)


_BM = 2000
_EB = 256
_EROWS = E // CHUNK


def _tc_edge_prep(row_raw, col_raw):
    def body(r_ref, c_ref, ro_ref, co_ref):
        i = pl.program_id(0)
        rr = jax.lax.broadcasted_iota(jnp.int32, (_EB, CHUNK), 0) + i * _EB
        cc = jax.lax.broadcasted_iota(jnp.int32, (_EB, CHUNK), 1)
        pos = rr * CHUNK + cc
        is_pad = rr >= _EROWS
        ro_ref[...] = jnp.where(is_pad, pos % N, r_ref[...])
        co_ref[...] = jnp.where(is_pad, N + pos % (N_ACC - N), c_ref[...])

    return pl.pallas_call(
        body,
        grid=(ROWS2D // _EB,),
        in_specs=[pl.BlockSpec((_EB, CHUNK), lambda i: (i, 0)),
                  pl.BlockSpec((_EB, CHUNK), lambda i: (i, 0))],
        out_specs=[pl.BlockSpec((_EB, CHUNK), lambda i: (i, 0)),
                   pl.BlockSpec((_EB, CHUNK), lambda i: (i, 0))],
        out_shape=[jax.ShapeDtypeStruct((ROWS2D, CHUNK), jnp.int32),
                   jax.ShapeDtypeStruct((ROWS2D, CHUNK), jnp.int32)],
    )(row_raw, col_raw)


def _tc_matmul(x, W):
    def body(x_ref, w_ref, o_ref):
        o_ref[...] = jnp.dot(x_ref[...], w_ref[...],
                             preferred_element_type=jnp.float32)

    return pl.pallas_call(
        body,
        grid=(N // _BM,),
        in_specs=[pl.BlockSpec((_BM, D), lambda i: (i, 0)),
                  pl.BlockSpec((D, D), lambda i: (0, 0))],
        out_specs=pl.BlockSpec((_BM, D), lambda i: (i, 0)),
        out_shape=jax.ShapeDtypeStruct((N, D), jnp.float32),
    )(x, W)


def _tc_fuse_relu_mm(parts, b, W):
    def body(p_ref, b_ref, w_ref, o_ref):
        z = p_ref[0] + p_ref[1] + b_ref[...]
        h = jnp.maximum(z, 0.0)
        o_ref[...] = jnp.dot(h, w_ref[...], preferred_element_type=jnp.float32)

    return pl.pallas_call(
        body,
        grid=(N // _BM,),
        in_specs=[pl.BlockSpec((NC, _BM, D), lambda i: (0, i, 0)),
                  pl.BlockSpec((1, D), lambda i: (0, 0)),
                  pl.BlockSpec((D, D), lambda i: (0, 0))],
        out_specs=pl.BlockSpec((_BM, D), lambda i: (i, 0)),
        out_shape=jax.ShapeDtypeStruct((N, D), jnp.float32),
    )(parts, b, W)


def _tc_fuse_log_softmax(parts, b):
    def body(p_ref, b_ref, o_ref):
        z = p_ref[0] + p_ref[1] + b_ref[...]
        m = jnp.max(z, axis=-1, keepdims=True)
        ez = jnp.exp(z - m)
        lse = jnp.log(jnp.sum(ez, axis=-1, keepdims=True)) + m
        o_ref[...] = z - lse

    return pl.pallas_call(
        body,
        grid=(N // _BM,),
        in_specs=[pl.BlockSpec((NC, _BM, D), lambda i: (0, i, 0)),
                  pl.BlockSpec((1, D), lambda i: (0, 0))],
        out_specs=pl.BlockSpec((_BM, D), lambda i: (i, 0)),
        out_shape=jax.ShapeDtypeStruct((N, D), jnp.float32),
    )(parts, b)


def kernel(x, edge_index, W1, b1, W2, b2):
    row2d, col2d = _tc_edge_prep(
        edge_index[0].reshape(_EROWS, CHUNK),
        edge_index[1].reshape(_EROWS, CHUNK))
    zslab = jnp.zeros((ZROWS, D), jnp.float32)

    h = _tc_matmul(x, W1)
    p1 = _sc_aggregate(h, row2d, col2d, zslab)
    h2 = _tc_fuse_relu_mm(p1, b1.reshape(1, D), W2)
    p2 = _sc_aggregate(h2, row2d, col2d, zslab)
    return _tc_fuse_log_softmax(p2, b2.reshape(1, D))


# speedup vs baseline: 1.0461x; 1.0461x over previous
import functools

import jax
import jax.numpy as jnp
from jax import lax
from jax.experimental import pallas as pl
from jax.experimental.pallas import tpu as pltpu
from jax.experimental.pallas import tpu_sc as plsc

N = 10000
E = 320000
D = 128

NC = 2
NS = 16
CHUNK = 128
CPT = 80
ROWS2D = NC * NS * CPT
E_PAD = ROWS2D * CHUNK
N_ACC = 10112
ZROWS = N_ACC // NS
NBUF = 2
NPH = 2
HC = CPT // NPH


def _sc_aggregate(h, row2d, col2d, zslab):
    mesh = plsc.VectorSubcoreMesh(core_axis_name="c", subcore_axis_name="s")

    @functools.partial(
        pl.kernel,
        out_type=jax.ShapeDtypeStruct((NC, N_ACC, D), jnp.float32),
        mesh=mesh,
        scratch_types=(
            [pltpu.VMEM_SHARED((N_ACC, D), jnp.float32)]
            + [pltpu.VMEM((HC, CHUNK), jnp.int32)] * 2
            + [pltpu.VMEM((CHUNK, D), jnp.float32)] * NBUF
            + [pltpu.SemaphoreType.DMA] * (NBUF + 1)
        ),
    )
    def agg(h_hbm, row_hbm, col_hbm, z_hbm, out_hbm, acc, row_v, col_v, *rest):
        bufs = rest[:NBUF]
        gs = rest[NBUF:2 * NBUF]
        zs = rest[2 * NBUF]
        c = lax.axis_index("c")
        s = lax.axis_index("s")
        base = (c * NS + s) * CPT
        pltpu.async_copy(z_hbm, acc.at[pl.ds(s * ZROWS, ZROWS)], zs)
        pltpu.async_copy(row_hbm.at[pl.ds(base, HC)], row_v, gs[0])
        pltpu.async_copy(col_hbm.at[pl.ds(base, HC)], col_v, gs[1])
        pltpu.make_async_copy(z_hbm, acc.at[pl.ds(s * ZROWS, ZROWS)], zs).wait()
        pltpu.make_async_copy(row_hbm.at[pl.ds(base, HC)], row_v, gs[0]).wait()
        pltpu.make_async_copy(col_hbm.at[pl.ds(base, HC)], col_v, gs[1]).wait()
        plsc.subcore_barrier()

        for p in range(NPH):
            if p > 0:
                pltpu.sync_copy(row_hbm.at[pl.ds(base + p * HC, HC)], row_v)
                pltpu.sync_copy(col_hbm.at[pl.ds(base + p * HC, HC)], col_v)

            for b in range(NBUF):
                pltpu.async_copy(h_hbm.at[row_v.at[b]], bufs[b], gs[b])

            @pl.loop(0, HC // NBUF)
            def _(i):
                jbase = i * NBUF
                for b in range(NBUF):
                    j = jbase + b
                    pltpu.make_async_copy(
                        h_hbm.at[row_v.at[j]], bufs[b], gs[b]).wait()
                    pltpu.sync_copy(bufs[b], acc.at[col_v.at[j]], add=True)
                    jn = jbase + NBUF + b

                    @pl.when(jn < HC)
                    def _(b=b, jn=jn):
                        pltpu.async_copy(h_hbm.at[row_v.at[jn]], bufs[b], gs[b])

        plsc.subcore_barrier()
        pltpu.sync_copy(acc.at[pl.ds(s * ZROWS, ZROWS)],
                        out_hbm.at[c, pl.ds(s * ZROWS, ZROWS)])

    return agg(h, row2d, col2d, zslab)


_BM = 2000
_EB = 256
_EROWS = E // CHUNK


_MB = N // (ROWS2D // _EB)


def _tc_prep_mm(ei3, x, W):
    def body(r_ref, c_ref, x_ref, w_ref, h_ref, ro_ref, co_ref):
        i = pl.program_id(0)
        h_ref[...] = jnp.dot(x_ref[...], w_ref[...],
                             preferred_element_type=jnp.float32)
        rr = jax.lax.broadcasted_iota(jnp.int32, (_EB, CHUNK), 0) + i * _EB
        cc = jax.lax.broadcasted_iota(jnp.int32, (_EB, CHUNK), 1)
        pos = rr * CHUNK + cc
        is_pad = rr >= _EROWS
        ro_ref[...] = jnp.where(is_pad, pos % N, r_ref[0])
        co_ref[...] = jnp.where(is_pad, N + pos % (N_ACC - N), c_ref[0])

    return pl.pallas_call(
        body,
        grid=(ROWS2D // _EB,),
        in_specs=[pl.BlockSpec((1, _EB, CHUNK), lambda i: (0, i, 0)),
                  pl.BlockSpec((1, _EB, CHUNK), lambda i: (1, i, 0)),
                  pl.BlockSpec((_MB, D), lambda i: (i, 0)),
                  pl.BlockSpec((D, D), lambda i: (0, 0))],
        out_specs=[pl.BlockSpec((_MB, D), lambda i: (i, 0)),
                   pl.BlockSpec((_EB, CHUNK), lambda i: (i, 0)),
                   pl.BlockSpec((_EB, CHUNK), lambda i: (i, 0))],
        out_shape=[jax.ShapeDtypeStruct((N, D), jnp.float32),
                   jax.ShapeDtypeStruct((ROWS2D, CHUNK), jnp.int32),
                   jax.ShapeDtypeStruct((ROWS2D, CHUNK), jnp.int32)],
    )(ei3, ei3, x, W)


def _tc_matmul(x, W):
    def body(x_ref, w_ref, o_ref):
        o_ref[...] = jnp.dot(x_ref[...], w_ref[...],
                             preferred_element_type=jnp.float32)

    return pl.pallas_call(
        body,
        grid=(N // _BM,),
        in_specs=[pl.BlockSpec((_BM, D), lambda i: (i, 0)),
                  pl.BlockSpec((D, D), lambda i: (0, 0))],
        out_specs=pl.BlockSpec((_BM, D), lambda i: (i, 0)),
        out_shape=jax.ShapeDtypeStruct((N, D), jnp.float32),
    )(x, W)


def _tc_fuse_relu_mm(parts, b, W):
    def body(p_ref, b_ref, w_ref, o_ref):
        z = p_ref[0] + p_ref[1] + b_ref[...]
        h = jnp.maximum(z, 0.0)
        o_ref[...] = jnp.dot(h, w_ref[...], preferred_element_type=jnp.float32)

    return pl.pallas_call(
        body,
        grid=(N // _BM,),
        in_specs=[pl.BlockSpec((NC, _BM, D), lambda i: (0, i, 0)),
                  pl.BlockSpec((1, D), lambda i: (0, 0)),
                  pl.BlockSpec((D, D), lambda i: (0, 0))],
        out_specs=pl.BlockSpec((_BM, D), lambda i: (i, 0)),
        out_shape=jax.ShapeDtypeStruct((N, D), jnp.float32),
    )(parts, b, W)


def _tc_fuse_log_softmax(parts, b):
    def body(p_ref, b_ref, o_ref):
        z = p_ref[0] + p_ref[1] + b_ref[...]
        m = jnp.max(z, axis=-1, keepdims=True)
        ez = jnp.exp(z - m)
        lse = jnp.log(jnp.sum(ez, axis=-1, keepdims=True)) + m
        o_ref[...] = z - lse

    return pl.pallas_call(
        body,
        grid=(N // _BM,),
        in_specs=[pl.BlockSpec((NC, _BM, D), lambda i: (0, i, 0)),
                  pl.BlockSpec((1, D), lambda i: (0, 0))],
        out_specs=pl.BlockSpec((_BM, D), lambda i: (i, 0)),
        out_shape=jax.ShapeDtypeStruct((N, D), jnp.float32),
    )(parts, b)


def kernel(x, edge_index, W1, b1, W2, b2):
    zslab = jnp.zeros((ZROWS, D), jnp.float32)
    h, row2d, col2d = _tc_prep_mm(
        edge_index.reshape(2, _EROWS, CHUNK), x, W1)
    p1 = _sc_aggregate(h, row2d, col2d, zslab)
    h2 = _tc_fuse_relu_mm(p1, b1.reshape(1, D), W2)
    p2 = _sc_aggregate(h2, row2d, col2d, zslab)
    return _tc_fuse_log_softmax(p2, b2.reshape(1, D))


# speedup vs baseline: 1.0482x; 1.0020x over previous
import functools

import jax
import jax.numpy as jnp
from jax import lax
from jax.experimental import pallas as pl
from jax.experimental.pallas import tpu as pltpu
from jax.experimental.pallas import tpu_sc as plsc

N = 10000
E = 320000
D = 128

NC = 2
NS = 16
CHUNK = 128
CPT = 80
ROWS2D = NC * NS * CPT
E_PAD = ROWS2D * CHUNK
N_ACC = 10112
ZROWS = N_ACC // NS
NBUF = 2
NPH = 2
HC = CPT // NPH


def _sc_aggregate(h, row2d, col2d, zslab):
    mesh = plsc.VectorSubcoreMesh(core_axis_name="c", subcore_axis_name="s")

    @functools.partial(
        pl.kernel,
        out_type=jax.ShapeDtypeStruct((NC, N_ACC, D), jnp.float32),
        mesh=mesh,
        scratch_types=(
            [pltpu.VMEM_SHARED((N_ACC, D), jnp.float32)]
            + [pltpu.VMEM((HC, CHUNK), jnp.int32)] * 2
            + [pltpu.VMEM((CHUNK, D), jnp.float32)] * NBUF
            + [pltpu.SemaphoreType.DMA] * (NBUF + 1)
        ),
    )
    def agg(h_hbm, row_hbm, col_hbm, z_hbm, out_hbm, acc, row_v, col_v, *rest):
        bufs = rest[:NBUF]
        gs = rest[NBUF:2 * NBUF]
        zs = rest[2 * NBUF]
        c = lax.axis_index("c")
        s = lax.axis_index("s")
        base = (c * NS + s) * CPT
        pltpu.async_copy(z_hbm, acc.at[pl.ds(s * ZROWS, ZROWS)], zs)
        pltpu.async_copy(row_hbm.at[pl.ds(base, HC)], row_v, gs[0])
        pltpu.async_copy(col_hbm.at[pl.ds(base, HC)], col_v, gs[1])
        pltpu.make_async_copy(z_hbm, acc.at[pl.ds(s * ZROWS, ZROWS)], zs).wait()
        pltpu.make_async_copy(row_hbm.at[pl.ds(base, HC)], row_v, gs[0]).wait()
        pltpu.make_async_copy(col_hbm.at[pl.ds(base, HC)], col_v, gs[1]).wait()
        plsc.subcore_barrier()

        for p in range(NPH):
            if p > 0:
                pltpu.sync_copy(row_hbm.at[pl.ds(base + p * HC, HC)], row_v)
                pltpu.sync_copy(col_hbm.at[pl.ds(base + p * HC, HC)], col_v)

            for b in range(NBUF):
                pltpu.async_copy(h_hbm.at[row_v.at[b]], bufs[b], gs[b])

            @pl.loop(0, HC // NBUF)
            def _(i):
                jbase = i * NBUF
                for b in range(NBUF):
                    j = jbase + b
                    pltpu.make_async_copy(
                        h_hbm.at[row_v.at[j]], bufs[b], gs[b]).wait()
                    pltpu.sync_copy(bufs[b], acc.at[col_v.at[j]], add=True)
                    jn = jbase + NBUF + b

                    @pl.when(jn < HC)
                    def _(b=b, jn=jn):
                        pltpu.async_copy(h_hbm.at[row_v.at[jn]], bufs[b], gs[b])

        plsc.subcore_barrier()
        pltpu.sync_copy(acc.at[pl.ds(s * ZROWS, ZROWS)],
                        out_hbm.at[c, pl.ds(s * ZROWS, ZROWS)])

    return agg(h, row2d, col2d, zslab)


_BM = 2000
_EB = 256
_EROWS = E // CHUNK


_MB = N // (ROWS2D // _EB)


def _tc_prep_mm(ei3, x, W):
    def body(r_ref, c_ref, x_ref, w_ref, h_ref, ro_ref, co_ref):
        i = pl.program_id(0)
        h_ref[...] = jnp.dot(x_ref[...], w_ref[...],
                             preferred_element_type=jnp.float32)
        rr = jax.lax.broadcasted_iota(jnp.int32, (_EB, CHUNK), 0) + i * _EB
        cc = jax.lax.broadcasted_iota(jnp.int32, (_EB, CHUNK), 1)
        pos = rr * CHUNK + cc
        is_pad = rr >= _EROWS
        ro_ref[...] = jnp.where(is_pad, pos % N, r_ref[0])
        co_ref[...] = jnp.where(is_pad, N + pos % (N_ACC - N), c_ref[0])

    return pl.pallas_call(
        body,
        grid=(ROWS2D // _EB,),
        in_specs=[pl.BlockSpec((1, _EB, CHUNK), lambda i: (0, i, 0)),
                  pl.BlockSpec((1, _EB, CHUNK), lambda i: (1, i, 0)),
                  pl.BlockSpec((_MB, D), lambda i: (i, 0)),
                  pl.BlockSpec((D, D), lambda i: (0, 0))],
        out_specs=[pl.BlockSpec((_MB, D), lambda i: (i, 0)),
                   pl.BlockSpec((_EB, CHUNK), lambda i: (i, 0)),
                   pl.BlockSpec((_EB, CHUNK), lambda i: (i, 0))],
        out_shape=[jax.ShapeDtypeStruct((N, D), jnp.float32),
                   jax.ShapeDtypeStruct((ROWS2D, CHUNK), jnp.int32),
                   jax.ShapeDtypeStruct((ROWS2D, CHUNK), jnp.int32)],
    )(ei3, ei3, x, W)


def _tc_fuse_relu_mm(parts, b, W):
    def body(p_ref, b_ref, w_ref, o_ref):
        z = p_ref[0] + p_ref[1] + b_ref[...]
        h = jnp.maximum(z, 0.0)
        o_ref[...] = jnp.dot(h, w_ref[...], preferred_element_type=jnp.float32)

    return pl.pallas_call(
        body,
        grid=(N // _BM,),
        in_specs=[pl.BlockSpec((NC, _BM, D), lambda i: (0, i, 0)),
                  pl.BlockSpec((1, D), lambda i: (0, 0)),
                  pl.BlockSpec((D, D), lambda i: (0, 0))],
        out_specs=pl.BlockSpec((_BM, D), lambda i: (i, 0)),
        out_shape=jax.ShapeDtypeStruct((N, D), jnp.float32),
    )(parts, b, W)


def _tc_fuse_log_softmax(parts, b):
    def body(p_ref, b_ref, o_ref):
        z = p_ref[0] + p_ref[1] + b_ref[...]
        m = jnp.max(z, axis=-1, keepdims=True)
        ez = jnp.exp(z - m)
        lse = jnp.log(jnp.sum(ez, axis=-1, keepdims=True)) + m
        o_ref[...] = z - lse

    return pl.pallas_call(
        body,
        grid=(N // _BM,),
        in_specs=[pl.BlockSpec((NC, _BM, D), lambda i: (0, i, 0)),
                  pl.BlockSpec((1, D), lambda i: (0, 0))],
        out_specs=pl.BlockSpec((_BM, D), lambda i: (i, 0)),
        out_shape=jax.ShapeDtypeStruct((N, D), jnp.float32),
    )(parts, b)


def kernel(x, edge_index, W1, b1, W2, b2):
    zslab = jnp.zeros((ZROWS, D), jnp.float32)
    h, row2d, col2d = _tc_prep_mm(
        edge_index.reshape(2, _EROWS, CHUNK), x, W1)
    p1 = _sc_aggregate(h, row2d, col2d, zslab)
    h2 = _tc_fuse_relu_mm(p1, b1.reshape(1, D), W2)
    p2 = _sc_aggregate(h2, row2d, col2d, zslab)
    return _tc_fuse_log_softmax(p2, b2.reshape(1, D))
